# Initial kernel scaffold; baseline (speedup 1.0000x reference)
import jax
import jax.numpy as jnp
from jax.experimental import pallas as pl


def kernel(rep, Dmat, bias, alpha, Z, mol_ids):
    raise NotImplementedError("write your pallas kernel here")


# speedup vs baseline: 2.3465x; 2.3465x over previous
import numpy as np
import jax
import jax.numpy as jnp
from jax.experimental import pallas as pl
from jax.experimental.pallas import tpu as pltpu

_N_ATOMS = 4096
_N_MOLS = 128
_N_ELEM = 4
_NSTACKS = 32
_NPCAS = 128
_SIGMA = 3.0
_NFEAT = _NSTACKS * _NPCAS

_B = 256
_NBLK = _N_ATOMS // _B


def _hadamard(n):
    H = np.array([[1.0]], dtype=np.float64)
    while H.shape[0] < n:
        H = np.block([[H, H], [H, -H]])
    return H


def _select4(z_col, tbl):
    r01 = jnp.where(z_col == 0, tbl[0:1, :], tbl[1:2, :])
    r23 = jnp.where(z_col == 2, tbl[2:3, :], tbl[3:4, :])
    return jnp.where(z_col <= 1, r01, r23)


_COEFF_NORM = np.float32(np.sqrt(np.float32(_NPCAS)) / _SIGMA)


def _tc_body(rep_ref, d0_ref, d1_ref, bias_ref, alpha_ref, hn_ref,
             z_ref, mol_ref, out_ref):
    z = z_ref[0, 0, :].reshape(_B, 1)
    mol = mol_ref[0, 0, :].reshape(_B, 1)
    rep = rep_ref[...]

    d0 = _select4(z, d0_ref[...])
    d1 = _select4(z, d1_ref[...])
    b = _select4(z, bias_ref[...])

    hn = hn_ref[...]
    v = (rep[:, None, :] * d0.reshape(_B, _NSTACKS, _NPCAS)).reshape(
        _B * _NSTACKS, _NPCAS)
    v = jax.lax.dot(v, hn)
    v = v * d1.reshape(_B * _NSTACKS, _NPCAS)
    v = jax.lax.dot(v, hn)

    feats = jnp.cos(_COEFF_NORM * v.reshape(_B, _NFEAT) + b)
    e = jnp.sum(feats * alpha_ref[...], axis=1)

    moh = (mol == jax.lax.broadcasted_iota(jnp.int32, (1, _N_MOLS), 1)
           ).astype(jnp.float32)
    contrib = jnp.sum(e[:, None] * moh, axis=0)

    @pl.when(pl.program_id(0) == 0)
    def _():
        out_ref[...] = jnp.zeros_like(out_ref)

    out_ref[...] += contrib[None, :]


def kernel(rep, Dmat, bias, alpha, Z, mol_ids):
    hn = jnp.asarray(_hadamard(_NPCAS) / np.sqrt(_NPCAS), dtype=jnp.float32)
    alpha_s = (alpha * np.float32(np.sqrt(2.0 / _NFEAT))).reshape(1, _NFEAT)

    d0 = Dmat[:, 0].reshape(_N_ELEM, _NFEAT)
    d1 = Dmat[:, 1].reshape(_N_ELEM, _NFEAT)
    z3 = Z.reshape(_NBLK, 1, _B)
    mol3 = mol_ids.reshape(_NBLK, 1, _B)

    out = pl.pallas_call(
        _tc_body,
        grid=(_NBLK,),
        in_specs=[
            pl.BlockSpec((_B, _NPCAS), lambda i: (i, 0)),
            pl.BlockSpec((_N_ELEM, _NFEAT), lambda i: (0, 0)),
            pl.BlockSpec((_N_ELEM, _NFEAT), lambda i: (0, 0)),
            pl.BlockSpec((_N_ELEM, _NFEAT), lambda i: (0, 0)),
            pl.BlockSpec((1, _NFEAT), lambda i: (0, 0)),
            pl.BlockSpec((_NPCAS, _NPCAS), lambda i: (0, 0)),
            pl.BlockSpec((1, 1, _B), lambda i: (i, 0, 0)),
            pl.BlockSpec((1, 1, _B), lambda i: (i, 0, 0)),
        ],
        out_specs=pl.BlockSpec((1, _N_MOLS), lambda i: (0, 0)),
        out_shape=jax.ShapeDtypeStruct((1, _N_MOLS), jnp.float32),
        compiler_params=pltpu.CompilerParams(
            dimension_semantics=("arbitrary",),
        ),
    )(rep, d0, d1, bias, alpha_s, hn, z3, mol3)
    return out.reshape(_N_MOLS)
